# baseline (speedup 1.0000x reference)
import functools
import math

import jax
import jax.numpy as jnp
from jax import lax
from jax.experimental import pallas as pl
from jax.experimental.pallas import tpu as pltpu

B, T, D = 4, 128, 512
LEVELS = 3
MAXK = 64
TT = 128
NTT = (B * T) // TT
_RSQRT_D = 1.0 / math.sqrt(D)


def _f32_key_u32(s):
    b = pltpu.bitcast(s, jnp.uint32)
    neg = (b >> 31) == jnp.uint32(1)
    return jnp.where(neg, ~b, b | jnp.uint32(0x80000000))


def _gelu(x):
    return 0.5 * x * (1.0 + lax.erf(x * (1.0 / math.sqrt(2.0))))


def _router_body(q_ref, rW1_ref, rb1_ref, rW2_ref, rb2_ref, pW1_ref, pb1_ref,
                 pW2_ref, pb2_ref, mtk_ref, rw_ref, dk_ref):
    q = q_ref[...]
    hi = jax.lax.Precision.DEFAULT
    h = _gelu(jnp.dot(q, rW1_ref[...], preferred_element_type=jnp.float32,
                      precision=hi) + rb1_ref[...])
    logits = jnp.dot(h, rW2_ref[...], preferred_element_type=jnp.float32,
                     precision=hi) + rb2_ref[...]
    col = lax.broadcasted_iota(jnp.int32, logits.shape, 1)
    logits = jnp.where(col < LEVELS, logits, -1e30)
    m = jnp.max(logits, axis=1, keepdims=True)
    e = jnp.exp(logits - m)
    rw_ref[...] = e / jnp.sum(e, axis=1, keepdims=True)

    p = _gelu(jnp.dot(q, pW1_ref[...], preferred_element_type=jnp.float32,
                      precision=hi) + pb1_ref[...])
    pb = p.astype(jnp.bfloat16).astype(jnp.float32)
    wb = pW2_ref[...].astype(jnp.bfloat16).astype(jnp.float32)
    pkl = jnp.sum(pb * wb, axis=1, keepdims=True) + pb2_ref[0, 0]
    pk = 1.0 / (1.0 + jnp.exp(-pkl))
    mtk = mtk_ref[0, 0].astype(jnp.float32)
    dk = jnp.clip((pk * mtk).astype(jnp.int32), 1, MAXK)
    dk_ref[...] = jnp.broadcast_to(dk, dk_ref.shape)


NSEG = 256
TOPC = 5
RSEL = 64
NSEL = (B * T) // RSEL


def _inv_key_f32(t):
    b = jnp.where((t >> 31) == jnp.uint32(1), t ^ jnp.uint32(0x80000000), ~t)
    return pltpu.bitcast(b, jnp.float32)


def _level_body(q_ref, k_ref, v_ref, sal_ref, dk_ref, rw_ref, out_ref,
                scores, wsum, *, nck, ck_sz, lvl, s_l):
    i = pl.program_id(0)

    @pl.when(i < nck)
    def _scores():
        s = jnp.dot(q_ref[...], k_ref[...].T, preferred_element_type=jnp.float32,
                    precision=jax.lax.Precision.DEFAULT) * _RSQRT_D
        scores[:, pl.ds(i * ck_sz, ck_sz)] = s + sal_ref[0]

    @pl.when((i >= nck) & (i < nck + NSEL))
    def _select():
        rows = pl.ds((i - nck) * RSEL, RSEL)
        s = scores[rows, :]
        dk = dk_ref[rows, 0:1]
        nsl = s_l // NSEG

        regs = [jnp.full((RSEL, NSEG), -jnp.inf, jnp.float32)
                for _ in range(TOPC)]
        for sl in range(nsl):
            v = s[:, sl * NSEG:(sl + 1) * NSEG]
            for j in range(TOPC):
                hi = jnp.maximum(regs[j], v)
                v = jnp.minimum(regs[j], v)
                regs[j] = hi
        m = jnp.max(regs[0], axis=1, keepdims=True)
        cand_arr = jnp.concatenate(regs, axis=1)
        u_cand = _f32_key_u32(cand_arr)

        t = jnp.zeros((RSEL, 1), jnp.uint32)
        for bit in range(31, -1, -1):
            cand = t | jnp.uint32(1 << bit)
            cnt = jnp.sum((u_cand >= cand).astype(jnp.int32), axis=1,
                          keepdims=True)
            t = jnp.where(cnt >= dk, cand, t)
        t_fast = _inv_key_f32(t)
        cnt_cand = jnp.sum((u_cand >= t).astype(jnp.int32), axis=1,
                           keepdims=True)
        ovf = jnp.sum((regs[TOPC - 1] >= t_fast).astype(jnp.int32), axis=1,
                      keepdims=True)
        wsum_fast = jnp.sum(
            jnp.where(u_cand >= t, jnp.exp(cand_arr - m), 0.0),
            axis=1, keepdims=True)
        ok = jnp.all((cnt_cand == dk) & (ovf == 0))

        def _slow():
            tu = jnp.zeros((RSEL, 1), jnp.uint32)
            for bit in range(31, -1, -1):
                cand = tu | jnp.uint32(1 << bit)
                fc = _inv_key_f32(cand)
                cnt = jnp.sum((s >= fc).astype(jnp.int32), axis=1,
                              keepdims=True)
                tu = jnp.where(cnt >= dk, cand, tu)
            ts = _inv_key_f32(tu)
            ws = jnp.sum(jnp.where(s >= ts, jnp.exp(s - m), 0.0),
                         axis=1, keepdims=True)
            return ts, ws

        t_f, ws = jax.lax.cond(ok, lambda: (t_fast, wsum_fast), _slow)
        w = jnp.where(s >= t_f, jnp.exp(s - m), 0.0)
        scores[rows, :] = w
        wsum[rows, :] = ws

    @pl.when(i >= nck + NSEL)
    def _read():
        vc = i - nck - NSEL
        a = jnp.dot(scores[:, pl.ds(vc * ck_sz, ck_sz)], v_ref[...],
                    preferred_element_type=jnp.float32)
        @pl.when(vc == 0)
        def _():
            out_ref[...] = a
        @pl.when(vc > 0)
        def _():
            out_ref[...] += a

        @pl.when(vc == nck - 1)
        def _():
            out_ref[...] = out_ref[...] * (rw_ref[:, lvl:lvl + 1] / wsum[...])


def _level_call(q2, K, V, sal, dkb, rw, lvl):
    s_l = K.shape[0]
    ck_sz = 512
    nck = s_l // ck_sz
    sal3 = sal.reshape(nck, 1, ck_sz)
    grid = (nck + NSEL + nck,)
    body = functools.partial(_level_body, nck=nck, ck_sz=ck_sz, lvl=lvl,
                             s_l=s_l)
    return pl.pallas_call(
        body,
        grid=grid,
        in_specs=[
            pl.BlockSpec((B * T, D), lambda i: (0, 0)),
            pl.BlockSpec((ck_sz, D), lambda i: (jnp.minimum(i, nck - 1), 0)),
            pl.BlockSpec((ck_sz, D), lambda i: (jnp.clip(i - (nck + NSEL), 0, nck - 1), 0)),
            pl.BlockSpec((1, 1, ck_sz), lambda i: (jnp.minimum(i, nck - 1), 0, 0)),
            pl.BlockSpec((B * T, 128), lambda i: (0, 0)),
            pl.BlockSpec((B * T, 128), lambda i: (0, 0)),
        ],
        out_specs=pl.BlockSpec((B * T, D), lambda i: (0, 0)),
        out_shape=jax.ShapeDtypeStruct((B * T, D), jnp.float32),
        scratch_shapes=[
            pltpu.VMEM((B * T, s_l), jnp.float32),
            pltpu.VMEM((B * T, 1), jnp.float32),
        ],
        compiler_params=pltpu.CompilerParams(
            dimension_semantics=("arbitrary",)),
    )(q2, K, V, sal3, dkb, rw)


def kernel(q, max_topk, K0, V0, sal0, K1, V1, sal1, K2, V2, sal2,
           rW1, rb1, rW2, rb2, pW1, pb1, pW2, pb2):
    q2 = q.reshape(B * T, D)
    rW2p = jnp.zeros((D, 128), jnp.float32).at[:, :LEVELS].set(rW2)
    rb2p = jnp.zeros((1, 128), jnp.float32).at[0, :LEVELS].set(rb2)
    pW1p = jnp.zeros((D, 128), jnp.float32).at[:, :64].set(pW1)
    pb1p = jnp.zeros((1, 128), jnp.float32).at[0, :64].set(pb1)
    pW2p = jnp.zeros((1, 128), jnp.float32).at[0, :64].set(pW2[:, 0])
    pb2p = jnp.full((1, 1), pb2[0], jnp.float32)
    mtk = jnp.full((1, 1), max_topk, jnp.int32)

    rw, dkb = pl.pallas_call(
        _router_body,
        grid=(NTT,),
        in_specs=[
            pl.BlockSpec((TT, D), lambda i: (i, 0)),
            pl.BlockSpec((D, D), lambda i: (0, 0)),
            pl.BlockSpec((1, D), lambda i: (0, 0)),
            pl.BlockSpec((D, 128), lambda i: (0, 0)),
            pl.BlockSpec((1, 128), lambda i: (0, 0)),
            pl.BlockSpec((D, 128), lambda i: (0, 0)),
            pl.BlockSpec((1, 128), lambda i: (0, 0)),
            pl.BlockSpec((1, 128), lambda i: (0, 0)),
            pl.BlockSpec((1, 1), lambda i: (0, 0), memory_space=pltpu.SMEM),
            pl.BlockSpec((1, 1), lambda i: (0, 0), memory_space=pltpu.SMEM),
        ],
        out_specs=[
            pl.BlockSpec((TT, 128), lambda i: (i, 0)),
            pl.BlockSpec((TT, 128), lambda i: (i, 0)),
        ],
        out_shape=[
            jax.ShapeDtypeStruct((B * T, 128), jnp.float32),
            jax.ShapeDtypeStruct((B * T, 128), jnp.int32),
        ],
    )(q2, rW1, rb1.reshape(1, D), rW2p, rb2p, pW1p, pb1p, pW2p, pb2p, mtk)

    r0 = _level_call(q2, K0, V0.astype(jnp.bfloat16), sal0, dkb, rw, 0)
    r1 = _level_call(q2, K1, V1.astype(jnp.bfloat16), sal1, dkb, rw, 1)
    r2 = _level_call(q2, K2, V2.astype(jnp.bfloat16), sal2, dkb, rw, 2)
    final_read = (r0 + r1 + r2).reshape(B, T, D)
    route_weights = rw[:, :LEVELS].reshape(B, T, LEVELS)
    return final_read, route_weights


# speedup vs baseline: 1.0329x; 1.0329x over previous
import functools
import math

import jax
import jax.numpy as jnp
from jax import lax
from jax.experimental import pallas as pl
from jax.experimental.pallas import tpu as pltpu

B, T, D = 4, 128, 512
LEVELS = 3
MAXK = 64
TT = 128
NTT = (B * T) // TT
_RSQRT_D = 1.0 / math.sqrt(D)


def _f32_key_u32(s):
    b = pltpu.bitcast(s, jnp.uint32)
    neg = (b >> 31) == jnp.uint32(1)
    return jnp.where(neg, ~b, b | jnp.uint32(0x80000000))


def _gelu(x):
    return 0.5 * x * (1.0 + lax.erf(x * (1.0 / math.sqrt(2.0))))


def _router_body(q_ref, rW1_ref, rb1_ref, rW2_ref, rb2_ref, pW1_ref, pb1_ref,
                 pW2_ref, pb2_ref, mtk_ref, rw_ref, dk_ref):
    q = q_ref[...]
    hi = jax.lax.Precision.DEFAULT
    h = _gelu(jnp.dot(q, rW1_ref[...], preferred_element_type=jnp.float32,
                      precision=hi) + rb1_ref[...])
    logits = jnp.dot(h, rW2_ref[...], preferred_element_type=jnp.float32,
                     precision=hi) + rb2_ref[...]
    col = lax.broadcasted_iota(jnp.int32, logits.shape, 1)
    logits = jnp.where(col < LEVELS, logits, -1e30)
    m = jnp.max(logits, axis=1, keepdims=True)
    e = jnp.exp(logits - m)
    rw_ref[...] = e / jnp.sum(e, axis=1, keepdims=True)

    p = _gelu(jnp.dot(q, pW1_ref[...], preferred_element_type=jnp.float32,
                      precision=hi) + pb1_ref[...])
    pb = p.astype(jnp.bfloat16).astype(jnp.float32)
    wb = pW2_ref[...].astype(jnp.bfloat16).astype(jnp.float32)
    pkl = jnp.sum(pb * wb, axis=1, keepdims=True) + pb2_ref[0, 0]
    pk = 1.0 / (1.0 + jnp.exp(-pkl))
    mtk = mtk_ref[0, 0].astype(jnp.float32)
    dk = jnp.clip((pk * mtk).astype(jnp.int32), 1, MAXK)
    dk_ref[...] = jnp.broadcast_to(dk, dk_ref.shape)


NSEG = 256
TOPC = 6
RSEL = 64
NSEL = (B * T) // RSEL


def _inv_key_f32(t):
    b = jnp.where((t >> 31) == jnp.uint32(1), t ^ jnp.uint32(0x80000000), ~t)
    return pltpu.bitcast(b, jnp.float32)


def _level_body(q_ref, k_ref, v_ref, sal_ref, dk_ref, rw_ref, out_ref,
                scores, wsum, *, nck, ck_sz, lvl, s_l):
    i = pl.program_id(0)

    @pl.when(i < nck)
    def _scores():
        s = jnp.dot(q_ref[...], k_ref[...].T, preferred_element_type=jnp.float32,
                    precision=jax.lax.Precision.DEFAULT) * _RSQRT_D
        scores[:, pl.ds(i * ck_sz, ck_sz)] = s + sal_ref[0]

    @pl.when((i >= nck) & (i < nck + NSEL))
    def _select():
        rows = pl.ds((i - nck) * RSEL, RSEL)
        s = scores[rows, :]
        dk = dk_ref[rows, 0:1]
        nsl = s_l // NSEG

        regs = [jnp.full((RSEL, NSEG), -jnp.inf, jnp.float32)
                for _ in range(TOPC)]
        for sl in range(nsl):
            v = s[:, sl * NSEG:(sl + 1) * NSEG]
            for j in range(TOPC):
                hi = jnp.maximum(regs[j], v)
                v = jnp.minimum(regs[j], v)
                regs[j] = hi
        m = jnp.max(regs[0], axis=1, keepdims=True)
        cand_arr = jnp.concatenate(regs, axis=1)
        u_cand = _f32_key_u32(cand_arr)

        t = jnp.zeros((RSEL, 1), jnp.uint32)
        for bit in range(31, -1, -1):
            cand = t | jnp.uint32(1 << bit)
            cnt = jnp.sum((u_cand >= cand).astype(jnp.int32), axis=1,
                          keepdims=True)
            t = jnp.where(cnt >= dk, cand, t)
        t_fast = _inv_key_f32(t)
        cnt_cand = jnp.sum((u_cand >= t).astype(jnp.int32), axis=1,
                           keepdims=True)
        ovf = jnp.sum((regs[TOPC - 1] >= t_fast).astype(jnp.int32), axis=1,
                      keepdims=True)
        wsum_fast = jnp.sum(
            jnp.where(u_cand >= t, jnp.exp(cand_arr - m), 0.0),
            axis=1, keepdims=True)
        ok = jnp.all((cnt_cand == dk) & (ovf == 0))

        def _slow():
            tu = jnp.zeros((RSEL, 1), jnp.uint32)
            for bit in range(31, -1, -1):
                cand = tu | jnp.uint32(1 << bit)
                fc = _inv_key_f32(cand)
                cnt = jnp.sum((s >= fc).astype(jnp.int32), axis=1,
                              keepdims=True)
                tu = jnp.where(cnt >= dk, cand, tu)
            ts = _inv_key_f32(tu)
            ws = jnp.sum(jnp.where(s >= ts, jnp.exp(s - m), 0.0),
                         axis=1, keepdims=True)
            return ts, ws

        t_f, ws = jax.lax.cond(ok, lambda: (t_fast, wsum_fast), _slow)
        w = jnp.where(s >= t_f, jnp.exp(s - m), 0.0)
        scores[rows, :] = w
        wsum[rows, :] = ws

    @pl.when(i >= nck + NSEL)
    def _read():
        vc = i - nck - NSEL
        a = jnp.dot(scores[:, pl.ds(vc * ck_sz, ck_sz)], v_ref[...],
                    preferred_element_type=jnp.float32)
        @pl.when(vc == 0)
        def _():
            out_ref[...] = a
        @pl.when(vc > 0)
        def _():
            out_ref[...] += a

        @pl.when(vc == nck - 1)
        def _():
            out_ref[...] = out_ref[...] * (rw_ref[:, lvl:lvl + 1] / wsum[...])


def _level_call(q2, K, V, sal, dkb, rw, lvl):
    s_l = K.shape[0]
    ck_sz = 512
    nck = s_l // ck_sz
    sal3 = sal.reshape(nck, 1, ck_sz)
    grid = (nck + NSEL + nck,)
    body = functools.partial(_level_body, nck=nck, ck_sz=ck_sz, lvl=lvl,
                             s_l=s_l)
    return pl.pallas_call(
        body,
        grid=grid,
        in_specs=[
            pl.BlockSpec((B * T, D), lambda i: (0, 0)),
            pl.BlockSpec((ck_sz, D), lambda i: (jnp.minimum(i, nck - 1), 0)),
            pl.BlockSpec((ck_sz, D), lambda i: (jnp.clip(i - (nck + NSEL), 0, nck - 1), 0)),
            pl.BlockSpec((1, 1, ck_sz), lambda i: (jnp.minimum(i, nck - 1), 0, 0)),
            pl.BlockSpec((B * T, 128), lambda i: (0, 0)),
            pl.BlockSpec((B * T, 128), lambda i: (0, 0)),
        ],
        out_specs=pl.BlockSpec((B * T, D), lambda i: (0, 0)),
        out_shape=jax.ShapeDtypeStruct((B * T, D), jnp.float32),
        scratch_shapes=[
            pltpu.VMEM((B * T, s_l), jnp.float32),
            pltpu.VMEM((B * T, 1), jnp.float32),
        ],
        compiler_params=pltpu.CompilerParams(
            dimension_semantics=("arbitrary",)),
    )(q2, K, V, sal3, dkb, rw)


def kernel(q, max_topk, K0, V0, sal0, K1, V1, sal1, K2, V2, sal2,
           rW1, rb1, rW2, rb2, pW1, pb1, pW2, pb2):
    q2 = q.reshape(B * T, D)
    rW2p = jnp.zeros((D, 128), jnp.float32).at[:, :LEVELS].set(rW2)
    rb2p = jnp.zeros((1, 128), jnp.float32).at[0, :LEVELS].set(rb2)
    pW1p = jnp.zeros((D, 128), jnp.float32).at[:, :64].set(pW1)
    pb1p = jnp.zeros((1, 128), jnp.float32).at[0, :64].set(pb1)
    pW2p = jnp.zeros((1, 128), jnp.float32).at[0, :64].set(pW2[:, 0])
    pb2p = jnp.full((1, 1), pb2[0], jnp.float32)
    mtk = jnp.full((1, 1), max_topk, jnp.int32)

    rw, dkb = pl.pallas_call(
        _router_body,
        grid=(NTT,),
        in_specs=[
            pl.BlockSpec((TT, D), lambda i: (i, 0)),
            pl.BlockSpec((D, D), lambda i: (0, 0)),
            pl.BlockSpec((1, D), lambda i: (0, 0)),
            pl.BlockSpec((D, 128), lambda i: (0, 0)),
            pl.BlockSpec((1, 128), lambda i: (0, 0)),
            pl.BlockSpec((D, 128), lambda i: (0, 0)),
            pl.BlockSpec((1, 128), lambda i: (0, 0)),
            pl.BlockSpec((1, 128), lambda i: (0, 0)),
            pl.BlockSpec((1, 1), lambda i: (0, 0), memory_space=pltpu.SMEM),
            pl.BlockSpec((1, 1), lambda i: (0, 0), memory_space=pltpu.SMEM),
        ],
        out_specs=[
            pl.BlockSpec((TT, 128), lambda i: (i, 0)),
            pl.BlockSpec((TT, 128), lambda i: (i, 0)),
        ],
        out_shape=[
            jax.ShapeDtypeStruct((B * T, 128), jnp.float32),
            jax.ShapeDtypeStruct((B * T, 128), jnp.int32),
        ],
    )(q2, rW1, rb1.reshape(1, D), rW2p, rb2p, pW1p, pb1p, pW2p, pb2p, mtk)

    r0 = _level_call(q2, K0, V0, sal0, dkb, rw, 0)
    r1 = _level_call(q2, K1, V1, sal1, dkb, rw, 1)
    r2 = _level_call(q2, K2, V2, sal2, dkb, rw, 2)
    final_read = (r0 + r1 + r2).reshape(B, T, D)
    route_weights = rw[:, :LEVELS].reshape(B, T, LEVELS)
    return final_read, route_weights


# speedup vs baseline: 1.1379x; 1.1016x over previous
import functools
import math

import jax
import jax.numpy as jnp
from jax import lax
from jax.experimental import pallas as pl
from jax.experimental.pallas import tpu as pltpu

B, T, D = 4, 128, 512
LEVELS = 3
MAXK = 64
TT = 128
NTT = (B * T) // TT
_RSQRT_D = 1.0 / math.sqrt(D)


def _f32_key_u32(s):
    b = pltpu.bitcast(s, jnp.uint32)
    neg = (b >> 31) == jnp.uint32(1)
    return jnp.where(neg, ~b, b | jnp.uint32(0x80000000))


def _gelu(x):
    return 0.5 * x * (1.0 + lax.erf(x * (1.0 / math.sqrt(2.0))))


def _router_body(q_ref, rW1_ref, rb1_ref, rW2_ref, rb2_ref, pW1_ref, pb1_ref,
                 pW2_ref, pb2_ref, mtk_ref, rw_ref, dk_ref):
    q = q_ref[...]
    hi = jax.lax.Precision.DEFAULT
    h = _gelu(jnp.dot(q, rW1_ref[...], preferred_element_type=jnp.float32,
                      precision=hi) + rb1_ref[...])
    logits = jnp.dot(h, rW2_ref[...], preferred_element_type=jnp.float32,
                     precision=hi) + rb2_ref[...]
    col = lax.broadcasted_iota(jnp.int32, logits.shape, 1)
    logits = jnp.where(col < LEVELS, logits, -1e30)
    m = jnp.max(logits, axis=1, keepdims=True)
    e = jnp.exp(logits - m)
    rw_ref[...] = e / jnp.sum(e, axis=1, keepdims=True)

    p = _gelu(jnp.dot(q, pW1_ref[...], preferred_element_type=jnp.float32,
                      precision=hi) + pb1_ref[...])
    pb = p.astype(jnp.bfloat16).astype(jnp.float32)
    wb = pW2_ref[...].astype(jnp.bfloat16).astype(jnp.float32)
    pkl = jnp.sum(pb * wb, axis=1, keepdims=True) + pb2_ref[0, 0]
    pk = 1.0 / (1.0 + jnp.exp(-pkl))
    mtk = mtk_ref[0, 0].astype(jnp.float32)
    dk = jnp.clip((pk * mtk).astype(jnp.int32), 1, MAXK)
    dk_ref[...] = jnp.broadcast_to(dk, dk_ref.shape)


NSEG = 256
TOPC = 6
RSEL = 64
NSEL = (B * T) // RSEL


def _inv_key_f32(t):
    b = jnp.where((t >> 31) == jnp.uint32(1), t ^ jnp.uint32(0x80000000), ~t)
    return pltpu.bitcast(b, jnp.float32)


def _level_body(q_ref, k_ref, v_ref, sal_ref, dk_ref, rw_ref, out_ref,
                scores, wsum, *, nck, ck_sz, lvl, s_l):
    i = pl.program_id(0)

    @pl.when(i < nck)
    def _scores():
        s = jnp.dot(q_ref[...], k_ref[...].T, preferred_element_type=jnp.float32,
                    precision=jax.lax.Precision.DEFAULT) * _RSQRT_D
        scores[:, pl.ds(i * ck_sz, ck_sz)] = s + sal_ref[0]

    @pl.when((i >= nck) & (i < nck + NSEL))
    def _select():
        rows = pl.ds((i - nck) * RSEL, RSEL)
        s = scores[rows, :]
        dk = dk_ref[rows, 0:1]
        nsl = s_l // NSEG

        regs = [jnp.full((RSEL, NSEG), -jnp.inf, jnp.float32)
                for _ in range(TOPC)]
        for sl in range(nsl):
            v = s[:, sl * NSEG:(sl + 1) * NSEG]
            for j in range(TOPC):
                hi = jnp.maximum(regs[j], v)
                v = jnp.minimum(regs[j], v)
                regs[j] = hi
        m = jnp.max(regs[0], axis=1, keepdims=True)
        cand_arr = jnp.concatenate(regs, axis=1)
        u_cand = _f32_key_u32(cand_arr)

        t = jnp.zeros((RSEL, 1), jnp.uint32)
        for bit in range(31, -1, -1):
            cand = t | jnp.uint32(1 << bit)
            cnt = jnp.sum((u_cand >= cand).astype(jnp.int32), axis=1,
                          keepdims=True)
            t = jnp.where(cnt >= dk, cand, t)
        t_fast = _inv_key_f32(t)
        cnt_cand = jnp.sum((u_cand >= t).astype(jnp.int32), axis=1,
                           keepdims=True)
        ovf = jnp.sum((regs[TOPC - 1] >= t_fast).astype(jnp.int32), axis=1,
                      keepdims=True)
        wsum_fast = jnp.sum(
            jnp.where(u_cand >= t, jnp.exp(cand_arr - m), 0.0),
            axis=1, keepdims=True)
        ok = jnp.all((cnt_cand == dk) & (ovf == 0))

        def _slow():
            tu = jnp.zeros((RSEL, 1), jnp.uint32)
            for bit in range(31, -1, -1):
                cand = tu | jnp.uint32(1 << bit)
                fc = _inv_key_f32(cand)
                cnt = jnp.sum((s >= fc).astype(jnp.int32), axis=1,
                              keepdims=True)
                tu = jnp.where(cnt >= dk, cand, tu)
            ts = _inv_key_f32(tu)
            ws = jnp.sum(jnp.where(s >= ts, jnp.exp(s - m), 0.0),
                         axis=1, keepdims=True)
            return ts, ws

        t_f, ws = jax.lax.cond(ok, lambda: (t_fast, wsum_fast), _slow)
        w = jnp.where(s >= t_f, jnp.exp(s - m), 0.0)
        scores[rows, :] = w
        wsum[rows, :] = ws

    @pl.when(i >= nck + NSEL)
    def _read():
        vc = i - nck - NSEL
        a = jnp.dot(scores[:, pl.ds(vc * ck_sz, ck_sz)], v_ref[...],
                    preferred_element_type=jnp.float32)
        @pl.when(vc == 0)
        def _():
            out_ref[...] = a
        @pl.when(vc > 0)
        def _():
            out_ref[...] += a

        @pl.when(vc == nck - 1)
        def _():
            out_ref[...] = out_ref[...] * (rw_ref[:, lvl:lvl + 1] / wsum[...])


def _level_call(q2, K, V, sal, dkb, rw, lvl):
    s_l = K.shape[0]
    ck_sz = 1024
    nck = s_l // ck_sz
    sal3 = sal.reshape(nck, 1, ck_sz)
    grid = (nck + NSEL + nck,)
    body = functools.partial(_level_body, nck=nck, ck_sz=ck_sz, lvl=lvl,
                             s_l=s_l)
    return pl.pallas_call(
        body,
        grid=grid,
        in_specs=[
            pl.BlockSpec((B * T, D), lambda i: (0, 0)),
            pl.BlockSpec((ck_sz, D), lambda i: (jnp.minimum(i, nck - 1), 0)),
            pl.BlockSpec((ck_sz, D), lambda i: (jnp.clip(i - (nck + NSEL), 0, nck - 1), 0)),
            pl.BlockSpec((1, 1, ck_sz), lambda i: (jnp.minimum(i, nck - 1), 0, 0)),
            pl.BlockSpec((B * T, 128), lambda i: (0, 0)),
            pl.BlockSpec((B * T, 128), lambda i: (0, 0)),
        ],
        out_specs=pl.BlockSpec((B * T, D), lambda i: (0, 0)),
        out_shape=jax.ShapeDtypeStruct((B * T, D), jnp.float32),
        scratch_shapes=[
            pltpu.VMEM((B * T, s_l), jnp.float32),
            pltpu.VMEM((B * T, 1), jnp.float32),
        ],
        compiler_params=pltpu.CompilerParams(
            dimension_semantics=("arbitrary",)),
    )(q2, K, V, sal3, dkb, rw)


def kernel(q, max_topk, K0, V0, sal0, K1, V1, sal1, K2, V2, sal2,
           rW1, rb1, rW2, rb2, pW1, pb1, pW2, pb2):
    q2 = q.reshape(B * T, D)
    rW2p = jnp.zeros((D, 128), jnp.float32).at[:, :LEVELS].set(rW2)
    rb2p = jnp.zeros((1, 128), jnp.float32).at[0, :LEVELS].set(rb2)
    pW1p = jnp.zeros((D, 128), jnp.float32).at[:, :64].set(pW1)
    pb1p = jnp.zeros((1, 128), jnp.float32).at[0, :64].set(pb1)
    pW2p = jnp.zeros((1, 128), jnp.float32).at[0, :64].set(pW2[:, 0])
    pb2p = jnp.full((1, 1), pb2[0], jnp.float32)
    mtk = jnp.full((1, 1), max_topk, jnp.int32)

    rw, dkb = pl.pallas_call(
        _router_body,
        grid=(NTT,),
        in_specs=[
            pl.BlockSpec((TT, D), lambda i: (i, 0)),
            pl.BlockSpec((D, D), lambda i: (0, 0)),
            pl.BlockSpec((1, D), lambda i: (0, 0)),
            pl.BlockSpec((D, 128), lambda i: (0, 0)),
            pl.BlockSpec((1, 128), lambda i: (0, 0)),
            pl.BlockSpec((D, 128), lambda i: (0, 0)),
            pl.BlockSpec((1, 128), lambda i: (0, 0)),
            pl.BlockSpec((1, 128), lambda i: (0, 0)),
            pl.BlockSpec((1, 1), lambda i: (0, 0), memory_space=pltpu.SMEM),
            pl.BlockSpec((1, 1), lambda i: (0, 0), memory_space=pltpu.SMEM),
        ],
        out_specs=[
            pl.BlockSpec((TT, 128), lambda i: (i, 0)),
            pl.BlockSpec((TT, 128), lambda i: (i, 0)),
        ],
        out_shape=[
            jax.ShapeDtypeStruct((B * T, 128), jnp.float32),
            jax.ShapeDtypeStruct((B * T, 128), jnp.int32),
        ],
    )(q2, rW1, rb1.reshape(1, D), rW2p, rb2p, pW1p, pb1p, pW2p, pb2p, mtk)

    r0 = _level_call(q2, K0, V0, sal0, dkb, rw, 0)
    r1 = _level_call(q2, K1, V1, sal1, dkb, rw, 1)
    r2 = _level_call(q2, K2, V2, sal2, dkb, rw, 2)
    final_read = (r0 + r1 + r2).reshape(B, T, D)
    route_weights = rw[:, :LEVELS].reshape(B, T, LEVELS)
    return final_read, route_weights


# speedup vs baseline: 1.3169x; 1.1574x over previous
import functools
import math

import jax
import jax.numpy as jnp
from jax import lax
from jax.experimental import pallas as pl
from jax.experimental.pallas import tpu as pltpu

B, T, D = 4, 128, 512
LEVELS = 3
MAXK = 64
TT = 128
NTT = (B * T) // TT
_RSQRT_D = 1.0 / math.sqrt(D)


def _f32_key_u32(s):
    b = pltpu.bitcast(s, jnp.uint32)
    neg = (b >> 31) == jnp.uint32(1)
    return jnp.where(neg, ~b, b | jnp.uint32(0x80000000))


def _gelu(x):
    return 0.5 * x * (1.0 + lax.erf(x * (1.0 / math.sqrt(2.0))))


def _router_body(q_ref, rW1_ref, rb1_ref, rW2_ref, rb2_ref, pW1_ref, pb1_ref,
                 pW2_ref, pb2_ref, mtk_ref, rw_ref, dk_ref):
    q = q_ref[...]
    hi = jax.lax.Precision.DEFAULT
    h = _gelu(jnp.dot(q, rW1_ref[...], preferred_element_type=jnp.float32,
                      precision=hi) + rb1_ref[...])
    logits = jnp.dot(h, rW2_ref[...], preferred_element_type=jnp.float32,
                     precision=hi) + rb2_ref[...]
    col = lax.broadcasted_iota(jnp.int32, logits.shape, 1)
    logits = jnp.where(col < LEVELS, logits, -1e30)
    m = jnp.max(logits, axis=1, keepdims=True)
    e = jnp.exp(logits - m)
    rw_ref[...] = e / jnp.sum(e, axis=1, keepdims=True)

    p = _gelu(jnp.dot(q, pW1_ref[...], preferred_element_type=jnp.float32,
                      precision=hi) + pb1_ref[...])
    pb = p.astype(jnp.bfloat16).astype(jnp.float32)
    wb = pW2_ref[...].astype(jnp.bfloat16).astype(jnp.float32)
    pkl = jnp.sum(pb * wb, axis=1, keepdims=True) + pb2_ref[0, 0]
    pk = 1.0 / (1.0 + jnp.exp(-pkl))
    mtk = mtk_ref[0, 0].astype(jnp.float32)
    dk = jnp.clip((pk * mtk).astype(jnp.int32), 1, MAXK)
    dk_ref[...] = jnp.broadcast_to(dk, dk_ref.shape)


NSEG = 256
TOPC = 6
RSEL = 64
NSEL = (B * T) // RSEL


def _inv_key_f32(t):
    b = jnp.where((t >> 31) == jnp.uint32(1), t ^ jnp.uint32(0x80000000), ~t)
    return pltpu.bitcast(b, jnp.float32)


def _level_body(q_ref, k_ref, v_ref, sal_ref, dk_ref, rw_ref, out_ref,
                scores, wsum, *, nck, ck_sz, lvl, s_l):
    i = pl.program_id(0)

    @pl.when(i < nck)
    def _scores():
        s = jnp.dot(q_ref[...], k_ref[...].T, preferred_element_type=jnp.float32,
                    precision=jax.lax.Precision.DEFAULT) * _RSQRT_D
        scores[:, pl.ds(i * ck_sz, ck_sz)] = s + sal_ref[0]

    @pl.when((i >= nck) & (i < nck + NSEL))
    def _select():
        rows = pl.ds((i - nck) * RSEL, RSEL)
        s = scores[rows, :]
        dk = dk_ref[rows, 0:1]
        nsl = s_l // NSEG

        regs = [jnp.full((RSEL, NSEG), -jnp.inf, jnp.float32)
                for _ in range(TOPC)]
        for sl in range(nsl):
            v = s[:, sl * NSEG:(sl + 1) * NSEG]
            for j in range(TOPC):
                hi = jnp.maximum(regs[j], v)
                v = jnp.minimum(regs[j], v)
                regs[j] = hi
        m = jnp.max(regs[0], axis=1, keepdims=True)
        cand_arr = jnp.concatenate(regs, axis=1)
        u_cand = _f32_key_u32(cand_arr)

        t = jnp.zeros((RSEL, 1), jnp.uint32)
        for b in range(30, -2, -2):
            cs = [t | jnp.uint32(d << b) for d in (3, 2, 1)]
            ns = [jnp.sum((u_cand >= c).astype(jnp.int32), axis=1,
                          keepdims=True) for c in cs]
            t = jnp.where(ns[0] >= dk, cs[0],
                          jnp.where(ns[1] >= dk, cs[1],
                                    jnp.where(ns[2] >= dk, cs[2], t)))
        t_fast = _inv_key_f32(t)
        cnt_cand = jnp.sum((u_cand >= t).astype(jnp.int32), axis=1,
                           keepdims=True)
        ovf = jnp.sum((regs[TOPC - 1] >= t_fast).astype(jnp.int32), axis=1,
                      keepdims=True)
        wsum_fast = jnp.sum(
            jnp.where(u_cand >= t, jnp.exp(cand_arr - m), 0.0),
            axis=1, keepdims=True)
        ok = jnp.all((cnt_cand == dk) & (ovf == 0))

        def _slow():
            tu = jnp.zeros((RSEL, 1), jnp.uint32)
            for bit in range(31, -1, -1):
                cand = tu | jnp.uint32(1 << bit)
                fc = _inv_key_f32(cand)
                cnt = jnp.sum((s >= fc).astype(jnp.int32), axis=1,
                              keepdims=True)
                tu = jnp.where(cnt >= dk, cand, tu)
            ts = _inv_key_f32(tu)
            ws = jnp.sum(jnp.where(s >= ts, jnp.exp(s - m), 0.0),
                         axis=1, keepdims=True)
            return ts, ws

        t_f, ws = jax.lax.cond(ok, lambda: (t_fast, wsum_fast), _slow)
        w = jnp.where(s >= t_f, jnp.exp(s - m), 0.0)
        scores[rows, :] = w
        wsum[rows, :] = ws

    @pl.when(i >= nck + NSEL)
    def _read():
        vc = i - nck - NSEL
        a = jnp.dot(scores[:, pl.ds(vc * ck_sz, ck_sz)], v_ref[...],
                    preferred_element_type=jnp.float32)
        @pl.when(vc == 0)
        def _():
            out_ref[...] = a
        @pl.when(vc > 0)
        def _():
            out_ref[...] += a

        @pl.when(vc == nck - 1)
        def _():
            out_ref[...] = out_ref[...] * (rw_ref[:, lvl:lvl + 1] / wsum[...])


def _level_call(q2, K, V, sal, dkb, rw, lvl):
    s_l = K.shape[0]
    ck_sz = 1024
    nck = s_l // ck_sz
    sal3 = sal.reshape(nck, 1, ck_sz)
    grid = (nck + NSEL + nck,)
    body = functools.partial(_level_body, nck=nck, ck_sz=ck_sz, lvl=lvl,
                             s_l=s_l)
    return pl.pallas_call(
        body,
        grid=grid,
        in_specs=[
            pl.BlockSpec((B * T, D), lambda i: (0, 0)),
            pl.BlockSpec((ck_sz, D), lambda i: (jnp.minimum(i, nck - 1), 0)),
            pl.BlockSpec((ck_sz, D), lambda i: (jnp.clip(i - (nck + NSEL), 0, nck - 1), 0)),
            pl.BlockSpec((1, 1, ck_sz), lambda i: (jnp.minimum(i, nck - 1), 0, 0)),
            pl.BlockSpec((B * T, 128), lambda i: (0, 0)),
            pl.BlockSpec((B * T, 128), lambda i: (0, 0)),
        ],
        out_specs=pl.BlockSpec((B * T, D), lambda i: (0, 0)),
        out_shape=jax.ShapeDtypeStruct((B * T, D), jnp.float32),
        scratch_shapes=[
            pltpu.VMEM((B * T, s_l), jnp.float32),
            pltpu.VMEM((B * T, 1), jnp.float32),
        ],
        compiler_params=pltpu.CompilerParams(
            dimension_semantics=("arbitrary",)),
    )(q2, K, V, sal3, dkb, rw)


def kernel(q, max_topk, K0, V0, sal0, K1, V1, sal1, K2, V2, sal2,
           rW1, rb1, rW2, rb2, pW1, pb1, pW2, pb2):
    q2 = q.reshape(B * T, D)
    rW2p = jnp.zeros((D, 128), jnp.float32).at[:, :LEVELS].set(rW2)
    rb2p = jnp.zeros((1, 128), jnp.float32).at[0, :LEVELS].set(rb2)
    pW1p = jnp.zeros((D, 128), jnp.float32).at[:, :64].set(pW1)
    pb1p = jnp.zeros((1, 128), jnp.float32).at[0, :64].set(pb1)
    pW2p = jnp.zeros((1, 128), jnp.float32).at[0, :64].set(pW2[:, 0])
    pb2p = jnp.full((1, 1), pb2[0], jnp.float32)
    mtk = jnp.full((1, 1), max_topk, jnp.int32)

    rw, dkb = pl.pallas_call(
        _router_body,
        grid=(NTT,),
        in_specs=[
            pl.BlockSpec((TT, D), lambda i: (i, 0)),
            pl.BlockSpec((D, D), lambda i: (0, 0)),
            pl.BlockSpec((1, D), lambda i: (0, 0)),
            pl.BlockSpec((D, 128), lambda i: (0, 0)),
            pl.BlockSpec((1, 128), lambda i: (0, 0)),
            pl.BlockSpec((D, 128), lambda i: (0, 0)),
            pl.BlockSpec((1, 128), lambda i: (0, 0)),
            pl.BlockSpec((1, 128), lambda i: (0, 0)),
            pl.BlockSpec((1, 1), lambda i: (0, 0), memory_space=pltpu.SMEM),
            pl.BlockSpec((1, 1), lambda i: (0, 0), memory_space=pltpu.SMEM),
        ],
        out_specs=[
            pl.BlockSpec((TT, 128), lambda i: (i, 0)),
            pl.BlockSpec((TT, 128), lambda i: (i, 0)),
        ],
        out_shape=[
            jax.ShapeDtypeStruct((B * T, 128), jnp.float32),
            jax.ShapeDtypeStruct((B * T, 128), jnp.int32),
        ],
    )(q2, rW1, rb1.reshape(1, D), rW2p, rb2p, pW1p, pb1p, pW2p, pb2p, mtk)

    r0 = _level_call(q2, K0, V0, sal0, dkb, rw, 0)
    r1 = _level_call(q2, K1, V1, sal1, dkb, rw, 1)
    r2 = _level_call(q2, K2, V2, sal2, dkb, rw, 2)
    final_read = (r0 + r1 + r2).reshape(B, T, D)
    route_weights = rw[:, :LEVELS].reshape(B, T, LEVELS)
    return final_read, route_weights
